# baseline (speedup 1.0000x reference)
import functools

import jax
import jax.numpy as jnp
from jax import lax
from jax.experimental import pallas as pl
from jax.experimental.pallas import tpu as pltpu
from jax.experimental.pallas import tpu_sc as plsc

N = 10000
E = 320000
D_IN = 128
HEADS = 4
C_OUT = 32
HC = HEADS * C_OUT
ED = 16

NC = 2
NS = 16
NW = NC * NS
EPW = E // NW
K = 40
NCHUNK = EPW // K
RPS = 624
TAIL = N - RPS * NS
TAIL0 = RPS * NS

DR = 1256
DRPS = 80
DTAIL = DR - DRPS * (NS - 1) + 0
DTAIL0 = DRPS * (NS - 1)
DTAIL = DR - DTAIL0

EB = 8000
NB = 2000


def _enc_body(attr_ref, ett_ref, w1_ref, b1_ref, w2_ref, b2_ref, lw_ref,
              ea_ref, esum_ref):
    blk = attr_ref[...]
    t0 = blk[:, 0:1]
    cond = t0 < 1.0
    ete = jnp.where(cond, ett_ref[0:1, :], ett_ref[1:2, :])
    feat = blk[:, 1:1 + ED] + ete
    h = jnp.maximum(jnp.dot(feat, w1_ref[...], preferred_element_type=jnp.float32)
                    + b1_ref[...], 0.0)
    enh = jnp.dot(h, w2_ref[...], preferred_element_type=jnp.float32) + b2_ref[...]
    ea_ref[...] = jnp.dot(enh, lw_ref[...], preferred_element_type=jnp.float32)
    s = jnp.sum(enh, axis=0, keepdims=True)

    @pl.when(pl.program_id(0) == 0)
    def _():
        esum_ref[...] = jnp.zeros_like(esum_ref)

    esum_ref[...] += jnp.broadcast_to(s, esum_ref.shape)


def _run_encoder(edge_attr, ett, w1, b1, w2, b2, lw):
    grid = (E // EB,)
    return pl.pallas_call(
        _enc_body,
        grid=grid,
        in_specs=[
            pl.BlockSpec((EB, 1 + ED), lambda i: (i, 0)),
            pl.BlockSpec((2, ED), lambda i: (0, 0)),
            pl.BlockSpec((ED, ED), lambda i: (0, 0)),
            pl.BlockSpec((1, ED), lambda i: (0, 0)),
            pl.BlockSpec((ED, ED), lambda i: (0, 0)),
            pl.BlockSpec((1, ED), lambda i: (0, 0)),
            pl.BlockSpec((ED, HC), lambda i: (0, 0)),
        ],
        out_specs=[
            pl.BlockSpec((EB, HC), lambda i: (i, 0)),
            pl.BlockSpec((8, ED), lambda i: (0, 0)),
        ],
        out_shape=[
            jax.ShapeDtypeStruct((E, HC), jnp.float32),
            jax.ShapeDtypeStruct((8, ED), jnp.float32),
        ],
    )(edge_attr, ett, w1, b1, w2, b2, lw)


def _proj_body(x_ref, wl_ref, wr_ref, xl_ref, xr_ref):
    x = x_ref[...]
    xl_ref[...] = jnp.dot(x, wl_ref[...], preferred_element_type=jnp.float32)
    xr_ref[...] = jnp.dot(x, wr_ref[...], preferred_element_type=jnp.float32)


def _run_proj(x, wl, wr):
    grid = (N // NB,)
    return pl.pallas_call(
        _proj_body,
        grid=grid,
        in_specs=[
            pl.BlockSpec((NB, D_IN), lambda i: (i, 0)),
            pl.BlockSpec((D_IN, HC), lambda i: (0, 0)),
            pl.BlockSpec((D_IN, HC), lambda i: (0, 0)),
        ],
        out_specs=[
            pl.BlockSpec((NB, HC), lambda i: (i, 0)),
            pl.BlockSpec((NB, HC), lambda i: (i, 0)),
        ],
        out_shape=[
            jax.ShapeDtypeStruct((N, HC), jnp.float32),
            jax.ShapeDtypeStruct((N, HC), jnp.float32),
        ],
    )(x, wl, wr)


def _take16(v, idx):
    return lax.gather(
        v, idx[:, None],
        dimension_numbers=lax.GatherDimensionNumbers(
            offset_dims=(), collapsed_slice_dims=(0,), start_index_map=(0,)),
        slice_sizes=(1,),
        mode=lax.GatherScatterMode.PROMISE_IN_BOUNDS)

def _sc_body(src_hbm, dst_hbm, xl_hbm, xr_hbm, ea_hbm, att_hbm, zacc_hbm,
             acc_out, den_out,
             srcA, dstA, drowA, srcB, dstB, drowB,
             xlA, xrA, xlB, xrB, ea_v, msg_v, den_v, att_v,
             acc_sh, den_sh, semA, semB):
    cid = lax.axis_index("c")
    sid = lax.axis_index("s")
    wid = sid * NC + cid

    pltpu.sync_copy(att_hbm, att_v)
    pltpu.sync_copy(zacc_hbm.at[pl.ds(sid * RPS, RPS)], acc_sh.at[pl.ds(sid * RPS, RPS)])

    @pl.when(sid == NS - 1)
    def _():
        pltpu.sync_copy(zacc_hbm.at[pl.ds(TAIL0, TAIL)], acc_sh.at[pl.ds(TAIL0, TAIL)])

    @pl.when(sid != NS - 1)
    def _():
        pltpu.sync_copy(zacc_hbm.at[pl.ds(0, DRPS)], den_sh.at[pl.ds(sid * DRPS, DRPS)])

    @pl.when(sid == NS - 1)
    def _():
        pltpu.sync_copy(zacc_hbm.at[pl.ds(0, DTAIL)], den_sh.at[pl.ds(DTAIL0, DTAIL)])

    plsc.subcore_barrier()

    attc = [att_v[0, pl.ds(c * 16, 16)] for c in range(8)]
    lane = lax.iota(jnp.int32, 16)
    _ones = jnp.full((16,), 1.0, jnp.float32)
    _zeros = jnp.full((16,), 0.0, jnp.float32)
    oh = [jnp.where(lane == h, _ones, _zeros) for h in range(HEADS)]
    base0 = wid * EPW

    def copy_idx(k, src_v, dst_v):
        base = base0 + k * K
        pltpu.sync_copy(src_hbm.at[pl.ds(base, K)], src_v)
        pltpu.sync_copy(dst_hbm.at[pl.ds(base, K)], dst_v)

    def fire(src_v, dst_v, xl_v, xr_v, sem):
        pltpu.async_copy(xl_hbm.at[src_v], xl_v, sem)
        pltpu.async_copy(xr_hbm.at[dst_v], xr_v, sem)

    def drain(xl_v, xr_v, sem):
        pltpu.make_async_copy(xl_hbm.at[pl.ds(0, K)], xl_v, sem).wait()
        pltpu.make_async_copy(xr_hbm.at[pl.ds(0, K)], xr_v, sem).wait()

    def compute_scatter(k, dst_v, drow_v, xl_v, xr_v):
        base = base0 + k * K
        pltpu.sync_copy(ea_hbm.at[pl.ds(base, K)], ea_v)
        dvs = [dst_v[pl.ds(o, 16)] for o in (0, 16, 24)]
        for o, dvv in zip((0, 16, 24), dvs):
            drow_v[pl.ds(o, 16)] = lax.shift_right_logical(dvv, 3)
        for e in range(K):
            gi, go = (0, 0) if e < 16 else ((1, 16) if e < 32 else (2, 24))
            dv_e = dvs[gi]
            xlc = [xl_v[e, pl.ds(c * 16, 16)] for c in range(8)]
            pv = []
            for h in range(HEADS):
                ha = None
                for j in range(2):
                    c = 2 * h + j
                    t = xlc[c] + xr_v[e, pl.ds(c * 16, 16)] + ea_v[e, pl.ds(c * 16, 16)]
                    t = jnp.maximum(t, 0.2 * t)
                    t = t * attc[c]
                    ha = t if ha is None else ha + t
                for m in (8, 4, 2, 1):
                    ha = ha + _take16(ha, lane ^ m)
                pv.append(jnp.exp(ha))
            for c in range(8):
                msg_v[e, pl.ds(c * 16, 16)] = xlc[c] * pv[c // 2]
            d = dv_e[e - go]
            den16 = oh[0] * pv[0] + oh[1] * pv[1] + oh[2] * pv[2] + oh[3] * pv[3]
            for c in range(8):
                den_v[e, pl.ds(c * 16, 16)] = _zeros
            den_v[e, pl.ds((d & 7) * 16, 16)] = den16
        pltpu.sync_copy(msg_v, acc_sh.at[dst_v], add=True)
        pltpu.sync_copy(den_v, den_sh.at[drow_v], add=True)

    copy_idx(0, srcA, dstA)
    fire(srcA, dstA, xlA, xrA, semA)

    def body(i, carry):
        ka = 2 * i
        kb = 2 * i + 1
        copy_idx(kb, srcB, dstB)
        fire(srcB, dstB, xlB, xrB, semB)
        drain(xlA, xrA, semA)
        compute_scatter(ka, dstA, drowA, xlA, xrA)
        kc = lax.rem(ka + 2, NCHUNK)
        copy_idx(kc, srcA, dstA)
        fire(srcA, dstA, xlA, xrA, semA)
        drain(xlB, xrB, semB)
        compute_scatter(kb, dstB, drowB, xlB, xrB)
        return carry

    lax.fori_loop(0, NCHUNK // 2, body, 0)
    drain(xlA, xrA, semA)
    plsc.subcore_barrier()

    pltpu.sync_copy(acc_sh.at[pl.ds(sid * RPS, RPS)],
                    acc_out.at[cid, pl.ds(sid * RPS, RPS)])

    @pl.when(sid == NS - 1)
    def _():
        pltpu.sync_copy(acc_sh.at[pl.ds(TAIL0, TAIL)], acc_out.at[cid, pl.ds(TAIL0, TAIL)])

    @pl.when(sid != NS - 1)
    def _():
        pltpu.sync_copy(den_sh.at[pl.ds(sid * DRPS, DRPS)],
                        den_out.at[cid, pl.ds(sid * DRPS, DRPS)])

    @pl.when(sid == NS - 1)
    def _():
        pltpu.sync_copy(den_sh.at[pl.ds(DTAIL0, DTAIL)], den_out.at[cid, pl.ds(DTAIL0, DTAIL)])


def _run_sc(src, dst, xl, xr, ea, att1):
    mesh = plsc.VectorSubcoreMesh(core_axis_name="c", subcore_axis_name="s")
    zacc = jnp.zeros((N, HC), jnp.float32)
    fn = functools.partial(
        pl.kernel,
        mesh=mesh,
        out_type=[
            jax.ShapeDtypeStruct((NC, N, HC), jnp.float32),
            jax.ShapeDtypeStruct((NC, DR, HC), jnp.float32),
        ],
        scratch_types=[
            pltpu.VMEM((K,), jnp.int32),
            pltpu.VMEM((K,), jnp.int32),
            pltpu.VMEM((K,), jnp.int32),
            pltpu.VMEM((K,), jnp.int32),
            pltpu.VMEM((K,), jnp.int32),
            pltpu.VMEM((K,), jnp.int32),
            pltpu.VMEM((K, HC), jnp.float32),
            pltpu.VMEM((K, HC), jnp.float32),
            pltpu.VMEM((K, HC), jnp.float32),
            pltpu.VMEM((K, HC), jnp.float32),
            pltpu.VMEM((K, HC), jnp.float32),
            pltpu.VMEM((K, HC), jnp.float32),
            pltpu.VMEM((K, HC), jnp.float32),
            pltpu.VMEM((1, HC), jnp.float32),
            pltpu.VMEM_SHARED((N, HC), jnp.float32),
            pltpu.VMEM_SHARED((DR, HC), jnp.float32),
            pltpu.SemaphoreType.DMA,
            pltpu.SemaphoreType.DMA,
        ],
    )(_sc_body)
    return fn(src, dst, xl, xr, ea, att1, zacc)


def _merge_body(acc0_ref, acc1_ref, den0_ref, den1_ref, xl_ref, xr_ref,
                esum_ref, lw_ref, att_ref, bias_ref, out_ref):
    s = esum_ref[0:1, :] * jnp.float32(1.0 / E)
    r0 = jnp.dot(s, lw_ref[...], preferred_element_type=jnp.float32)
    xl = xl_ref[...]
    t = xl + xr_ref[...] + r0
    t = jnp.maximum(t, 0.2 * t)
    ta = t * att_ref[...]
    i0 = lax.broadcasted_iota(jnp.int32, (HC, HEADS), 0) // C_OUT
    h0 = lax.broadcasted_iota(jnp.int32, (HC, HEADS), 1)
    sel = (i0 == h0).astype(jnp.float32)
    i1 = lax.broadcasted_iota(jnp.int32, (HEADS, HC), 0)
    h1 = lax.broadcasted_iota(jnp.int32, (HEADS, HC), 1) // C_OUT
    selT = (i1 == h1).astype(jnp.float32)
    a4 = jnp.dot(ta, sel, preferred_element_type=jnp.float32)
    p4 = jnp.exp(a4)
    p128 = jnp.dot(p4, selT, preferred_element_type=jnp.float32)
    num = acc0_ref[...] + acc1_ref[...] + p128 * xl
    d4 = den0_ref[...] + den1_ref[...] + p4
    d128 = jnp.dot(d4, selT, preferred_element_type=jnp.float32)
    out_ref[...] = num / (d128 + 1e-16) + bias_ref[...]


def _run_merge(acc0, acc1, den0, den1, xl, xr, esum, lw, att2, bias2):
    grid = (N // NB,)
    return pl.pallas_call(
        _merge_body,
        grid=grid,
        in_specs=[
            pl.BlockSpec((NB, HC), lambda i: (i, 0)),
            pl.BlockSpec((NB, HC), lambda i: (i, 0)),
            pl.BlockSpec((NB, HEADS), lambda i: (i, 0)),
            pl.BlockSpec((NB, HEADS), lambda i: (i, 0)),
            pl.BlockSpec((NB, HC), lambda i: (i, 0)),
            pl.BlockSpec((NB, HC), lambda i: (i, 0)),
            pl.BlockSpec((8, ED), lambda i: (0, 0)),
            pl.BlockSpec((ED, HC), lambda i: (0, 0)),
            pl.BlockSpec((1, HC), lambda i: (0, 0)),
            pl.BlockSpec((1, HC), lambda i: (0, 0)),
        ],
        out_specs=pl.BlockSpec((NB, HC), lambda i: (i, 0)),
        out_shape=jax.ShapeDtypeStruct((N, HC), jnp.float32),
    )(acc0, acc1, den0, den1, xl, xr, esum, lw, att2, bias2)


def kernel(x, edge_index, edge_attr, edge_type_table, enc_W1, enc_b1, enc_W2, enc_b2,
           W_l, W_r, lin_edge_W, att, bias):
    ea, esum = _run_encoder(edge_attr, edge_type_table,
                            enc_W1, enc_b1.reshape(1, ED),
                            enc_W2, enc_b2.reshape(1, ED), lin_edge_W)
    xl, xr = _run_proj(x, W_l, W_r)
    src = edge_index[0]
    dst = edge_index[1]
    acc, den = _run_sc(src, dst, xl, xr, ea, att.reshape(1, HC))
    d4 = den[:, :N // 8].reshape(NC, N // 8, 8, 16)[..., :HEADS].reshape(NC, N, HEADS)
    out = _run_merge(acc[0], acc[1], d4[0], d4[1], xl, xr, esum,
                     lin_edge_W, att.reshape(1, HC), bias.reshape(1, HC))
    return out


# speedup vs baseline: 1.1007x; 1.1007x over previous
import functools

import jax
import jax.numpy as jnp
from jax import lax
from jax.experimental import pallas as pl
from jax.experimental.pallas import tpu as pltpu
from jax.experimental.pallas import tpu_sc as plsc

N = 10000
E = 320000
D_IN = 128
HEADS = 4
C_OUT = 32
HC = HEADS * C_OUT
ED = 16

NC = 2
NS = 16
NW = NC * NS
EPW = E // NW
K = 40
NCHUNK = EPW // K
RPS = 624
TAIL = N - RPS * NS
TAIL0 = RPS * NS

DR = 1256
DRPS = 80
DTAIL = DR - DRPS * (NS - 1) + 0
DTAIL0 = DRPS * (NS - 1)
DTAIL = DR - DTAIL0

EB = 8000
NB = 2000


def _enc_body(attr_ref, ett_ref, w1_ref, b1_ref, w2_ref, b2_ref, lw_ref,
              ea_ref, esum_ref):
    blk = attr_ref[...]
    t0 = blk[:, 0:1]
    cond = t0 < 1.0
    ete = jnp.where(cond, ett_ref[0:1, :], ett_ref[1:2, :])
    feat = blk[:, 1:1 + ED] + ete
    h = jnp.maximum(jnp.dot(feat, w1_ref[...], preferred_element_type=jnp.float32)
                    + b1_ref[...], 0.0)
    enh = jnp.dot(h, w2_ref[...], preferred_element_type=jnp.float32) + b2_ref[...]
    ea_ref[...] = jnp.dot(enh, lw_ref[...], preferred_element_type=jnp.float32)
    s = jnp.sum(enh, axis=0, keepdims=True)

    @pl.when(pl.program_id(0) == 0)
    def _():
        esum_ref[...] = jnp.zeros_like(esum_ref)

    esum_ref[...] += jnp.broadcast_to(s, esum_ref.shape)


def _run_encoder(edge_attr, ett, w1, b1, w2, b2, lw):
    grid = (E // EB,)
    return pl.pallas_call(
        _enc_body,
        grid=grid,
        in_specs=[
            pl.BlockSpec((EB, 1 + ED), lambda i: (i, 0)),
            pl.BlockSpec((2, ED), lambda i: (0, 0)),
            pl.BlockSpec((ED, ED), lambda i: (0, 0)),
            pl.BlockSpec((1, ED), lambda i: (0, 0)),
            pl.BlockSpec((ED, ED), lambda i: (0, 0)),
            pl.BlockSpec((1, ED), lambda i: (0, 0)),
            pl.BlockSpec((ED, HC), lambda i: (0, 0)),
        ],
        out_specs=[
            pl.BlockSpec((EB, HC), lambda i: (i, 0)),
            pl.BlockSpec((8, ED), lambda i: (0, 0)),
        ],
        out_shape=[
            jax.ShapeDtypeStruct((E, HC), jnp.float32),
            jax.ShapeDtypeStruct((8, ED), jnp.float32),
        ],
    )(edge_attr, ett, w1, b1, w2, b2, lw)


def _proj_body(x_ref, wl_ref, wr_ref, xl_ref, xr_ref):
    x = x_ref[...]
    xl_ref[...] = jnp.dot(x, wl_ref[...], preferred_element_type=jnp.float32)
    xr_ref[...] = jnp.dot(x, wr_ref[...], preferred_element_type=jnp.float32)


def _run_proj(x, wl, wr):
    grid = (N // NB,)
    return pl.pallas_call(
        _proj_body,
        grid=grid,
        in_specs=[
            pl.BlockSpec((NB, D_IN), lambda i: (i, 0)),
            pl.BlockSpec((D_IN, HC), lambda i: (0, 0)),
            pl.BlockSpec((D_IN, HC), lambda i: (0, 0)),
        ],
        out_specs=[
            pl.BlockSpec((NB, HC), lambda i: (i, 0)),
            pl.BlockSpec((NB, HC), lambda i: (i, 0)),
        ],
        out_shape=[
            jax.ShapeDtypeStruct((N, HC), jnp.float32),
            jax.ShapeDtypeStruct((N, HC), jnp.float32),
        ],
    )(x, wl, wr)


def _take16(v, idx):
    return lax.gather(
        v, idx[:, None],
        dimension_numbers=lax.GatherDimensionNumbers(
            offset_dims=(), collapsed_slice_dims=(0,), start_index_map=(0,)),
        slice_sizes=(1,),
        mode=lax.GatherScatterMode.PROMISE_IN_BOUNDS)

def _sc_body(src_hbm, dst_hbm, xl_hbm, xr_hbm, ea_hbm, att_hbm, zacc_hbm,
             acc_out, den_out,
             src_v, dst_v, dstrow_v, xl_v, xr_v, ea_v, msg_v, den_v, att_v,
             acc_sh, den_sh, sem):
    cid = lax.axis_index("c")
    sid = lax.axis_index("s")
    wid = sid * NC + cid

    pltpu.sync_copy(att_hbm, att_v)
    pltpu.sync_copy(zacc_hbm.at[pl.ds(sid * RPS, RPS)], acc_sh.at[pl.ds(sid * RPS, RPS)])

    @pl.when(sid == NS - 1)
    def _():
        pltpu.sync_copy(zacc_hbm.at[pl.ds(TAIL0, TAIL)], acc_sh.at[pl.ds(TAIL0, TAIL)])

    @pl.when(sid != NS - 1)
    def _():
        pltpu.sync_copy(zacc_hbm.at[pl.ds(0, DRPS)], den_sh.at[pl.ds(sid * DRPS, DRPS)])

    @pl.when(sid == NS - 1)
    def _():
        pltpu.sync_copy(zacc_hbm.at[pl.ds(0, DTAIL)], den_sh.at[pl.ds(DTAIL0, DTAIL)])

    plsc.subcore_barrier()

    attc = [att_v[0, pl.ds(c * 16, 16)] for c in range(8)]
    lane = lax.iota(jnp.int32, 16)
    _zeros = jnp.full((16,), 0.0, jnp.float32)
    _i0 = jnp.full((16,), 0, jnp.int32)
    _i8 = jnp.full((16,), 8, jnp.int32)
    _alt = (lane & 1) * 8
    base0 = wid * EPW

    for e0 in range(K):
        for c0 in range(8):
            den_v[e0, pl.ds(c0 * 16, 16)] = _zeros

    def chunk_body(k, carry):
        base = base0 + k * K
        pltpu.sync_copy(src_hbm.at[pl.ds(base, K)], src_v)
        pltpu.sync_copy(dst_hbm.at[pl.ds(base, K)], dst_v)
        pltpu.sync_copy(ea_hbm.at[pl.ds(base, K)], ea_v)
        c1 = pltpu.async_copy(xl_hbm.at[src_v], xl_v, sem)
        c2 = pltpu.async_copy(xr_hbm.at[dst_v], xr_v, sem)
        c1.wait()
        c2.wait()

        dvs = [dst_v[pl.ds(o, 16)] for o in (0, 16, 24)]
        for o, dvv in zip((0, 16, 24), dvs):
            dstrow_v[pl.ds(o, 16)] = lax.shift_right_logical(dvv, 3)
        offs = []
        for e in range(K):
            gi, go = (0, 0) if e < 16 else ((1, 16) if e < 32 else (2, 24))
            dv_e = dvs[gi]
            xlc = [xl_v[e, pl.ds(c * 16, 16)] for c in range(8)]
            hv = []
            for h in range(HEADS):
                ha = None
                for j in range(2):
                    c = 2 * h + j
                    t = xlc[c] + xr_v[e, pl.ds(c * 16, 16)] + ea_v[e, pl.ds(c * 16, 16)]
                    t = jnp.maximum(t, 0.2 * t)
                    t = t * attc[c]
                    ha = t if ha is None else ha + t
                hv.append(ha + _take16(ha, lane ^ 8))
            w01 = jnp.where(lane < 8, hv[0], hv[1])
            w23 = jnp.where(lane < 8, hv[2], hv[3])
            for m in (4, 2, 1):
                w01 = w01 + _take16(w01, lane ^ m)
                w23 = w23 + _take16(w23, lane ^ m)
            ep01 = jnp.exp(w01)
            ep23 = jnp.exp(w23)
            pv = [_take16(ep01, _i0), _take16(ep01, _i8),
                  _take16(ep23, _i0), _take16(ep23, _i8)]
            for c in range(8):
                msg_v[e, pl.ds(c * 16, 16)] = xlc[c] * pv[c // 2]
            d = dv_e[e - go]
            d01 = _take16(ep01, _alt)
            d23 = _take16(ep23, _alt)
            den16 = jnp.where(lane < 4, jnp.where(lane < 2, d01, d23), _zeros)
            off = (d & 7) * 16
            offs.append(off)
            den_v[e, pl.ds(off, 16)] = den16
        pltpu.sync_copy(msg_v, acc_sh.at[dst_v], add=True)
        pltpu.sync_copy(den_v, den_sh.at[dstrow_v], add=True)
        for e in range(K):
            den_v[e, pl.ds(offs[e], 16)] = _zeros
        return carry

    lax.fori_loop(0, NCHUNK, chunk_body, 0)
    plsc.subcore_barrier()

    pltpu.sync_copy(acc_sh.at[pl.ds(sid * RPS, RPS)],
                    acc_out.at[cid, pl.ds(sid * RPS, RPS)])

    @pl.when(sid == NS - 1)
    def _():
        pltpu.sync_copy(acc_sh.at[pl.ds(TAIL0, TAIL)], acc_out.at[cid, pl.ds(TAIL0, TAIL)])

    @pl.when(sid != NS - 1)
    def _():
        pltpu.sync_copy(den_sh.at[pl.ds(sid * DRPS, DRPS)],
                        den_out.at[cid, pl.ds(sid * DRPS, DRPS)])

    @pl.when(sid == NS - 1)
    def _():
        pltpu.sync_copy(den_sh.at[pl.ds(DTAIL0, DTAIL)], den_out.at[cid, pl.ds(DTAIL0, DTAIL)])


def _run_sc(src, dst, xl, xr, ea, att1):
    mesh = plsc.VectorSubcoreMesh(core_axis_name="c", subcore_axis_name="s")
    zacc = jnp.zeros((N, HC), jnp.float32)
    fn = functools.partial(
        pl.kernel,
        mesh=mesh,
        out_type=[
            jax.ShapeDtypeStruct((NC, N, HC), jnp.float32),
            jax.ShapeDtypeStruct((NC, DR, HC), jnp.float32),
        ],
        scratch_types=[
            pltpu.VMEM((K,), jnp.int32),
            pltpu.VMEM((K,), jnp.int32),
            pltpu.VMEM((K,), jnp.int32),
            pltpu.VMEM((K, HC), jnp.float32),
            pltpu.VMEM((K, HC), jnp.float32),
            pltpu.VMEM((K, HC), jnp.float32),
            pltpu.VMEM((K, HC), jnp.float32),
            pltpu.VMEM((K, HC), jnp.float32),
            pltpu.VMEM((1, HC), jnp.float32),
            pltpu.VMEM_SHARED((N, HC), jnp.float32),
            pltpu.VMEM_SHARED((DR, HC), jnp.float32),
            pltpu.SemaphoreType.DMA,
        ],
    )(_sc_body)
    return fn(src, dst, xl, xr, ea, att1, zacc)


def _merge_body(acc0_ref, acc1_ref, den0_ref, den1_ref, xl_ref, xr_ref,
                esum_ref, lw_ref, att_ref, bias_ref, out_ref):
    s = esum_ref[0:1, :] * jnp.float32(1.0 / E)
    r0 = jnp.dot(s, lw_ref[...], preferred_element_type=jnp.float32)
    xl = xl_ref[...]
    t = xl + xr_ref[...] + r0
    t = jnp.maximum(t, 0.2 * t)
    ta = t * att_ref[...]
    i0 = lax.broadcasted_iota(jnp.int32, (HC, HEADS), 0) // C_OUT
    h0 = lax.broadcasted_iota(jnp.int32, (HC, HEADS), 1)
    sel = (i0 == h0).astype(jnp.float32)
    i1 = lax.broadcasted_iota(jnp.int32, (HEADS, HC), 0)
    h1 = lax.broadcasted_iota(jnp.int32, (HEADS, HC), 1) // C_OUT
    selT = (i1 == h1).astype(jnp.float32)
    a4 = jnp.dot(ta, sel, preferred_element_type=jnp.float32)
    p4 = jnp.exp(a4)
    p128 = jnp.dot(p4, selT, preferred_element_type=jnp.float32)
    num = acc0_ref[...] + acc1_ref[...] + p128 * xl
    d4 = den0_ref[...] + den1_ref[...] + p4
    d128 = jnp.dot(d4, selT, preferred_element_type=jnp.float32)
    out_ref[...] = num / (d128 + 1e-16) + bias_ref[...]


def _run_merge(acc0, acc1, den0, den1, xl, xr, esum, lw, att2, bias2):
    grid = (N // NB,)
    return pl.pallas_call(
        _merge_body,
        grid=grid,
        in_specs=[
            pl.BlockSpec((NB, HC), lambda i: (i, 0)),
            pl.BlockSpec((NB, HC), lambda i: (i, 0)),
            pl.BlockSpec((NB, HEADS), lambda i: (i, 0)),
            pl.BlockSpec((NB, HEADS), lambda i: (i, 0)),
            pl.BlockSpec((NB, HC), lambda i: (i, 0)),
            pl.BlockSpec((NB, HC), lambda i: (i, 0)),
            pl.BlockSpec((8, ED), lambda i: (0, 0)),
            pl.BlockSpec((ED, HC), lambda i: (0, 0)),
            pl.BlockSpec((1, HC), lambda i: (0, 0)),
            pl.BlockSpec((1, HC), lambda i: (0, 0)),
        ],
        out_specs=pl.BlockSpec((NB, HC), lambda i: (i, 0)),
        out_shape=jax.ShapeDtypeStruct((N, HC), jnp.float32),
    )(acc0, acc1, den0, den1, xl, xr, esum, lw, att2, bias2)


def kernel(x, edge_index, edge_attr, edge_type_table, enc_W1, enc_b1, enc_W2, enc_b2,
           W_l, W_r, lin_edge_W, att, bias):
    ea, esum = _run_encoder(edge_attr, edge_type_table,
                            enc_W1, enc_b1.reshape(1, ED),
                            enc_W2, enc_b2.reshape(1, ED), lin_edge_W)
    xl, xr = _run_proj(x, W_l, W_r)
    src = edge_index[0]
    dst = edge_index[1]
    acc, den = _run_sc(src, dst, xl, xr, ea, att.reshape(1, HC))
    d4 = den[:, :N // 8].reshape(NC, N // 8, 8, 16)[..., :HEADS].reshape(NC, N, HEADS)
    out = _run_merge(acc[0], acc[1], d4[0], d4[1], xl, xr, esum,
                     lin_edge_W, att.reshape(1, HC), bias.reshape(1, HC))
    return out


# speedup vs baseline: 1.2394x; 1.1260x over previous
import functools

import jax
import jax.numpy as jnp
from jax import lax
from jax.experimental import pallas as pl
from jax.experimental.pallas import tpu as pltpu
from jax.experimental.pallas import tpu_sc as plsc

N = 10000
E = 320000
D_IN = 128
HEADS = 4
C_OUT = 32
HC = HEADS * C_OUT
ED = 16

NC = 2
NS = 16
NW = NC * NS
EPW = E // NW
K = 40
NCHUNK = EPW // K
RPS = 624
TAIL = N - RPS * NS
TAIL0 = RPS * NS

DR = 1256
DRPS = 80
DTAIL = DR - DRPS * (NS - 1) + 0
DTAIL0 = DRPS * (NS - 1)
DTAIL = DR - DTAIL0

EB = 8000
NB = 2000


def _enc_body(attr_ref, ett_ref, w1_ref, b1_ref, w2_ref, b2_ref, lw_ref,
              ea_ref, esum_ref):
    blk = attr_ref[...]
    t0 = blk[:, 0:1]
    cond = t0 < 1.0
    ete = jnp.where(cond, ett_ref[0:1, :], ett_ref[1:2, :])
    feat = blk[:, 1:1 + ED] + ete
    h = jnp.maximum(jnp.dot(feat, w1_ref[...], preferred_element_type=jnp.float32)
                    + b1_ref[...], 0.0)
    enh = jnp.dot(h, w2_ref[...], preferred_element_type=jnp.float32) + b2_ref[...]
    ea_ref[...] = jnp.dot(enh, lw_ref[...], preferred_element_type=jnp.float32)
    s = jnp.sum(enh, axis=0, keepdims=True)

    @pl.when(pl.program_id(0) == 0)
    def _():
        esum_ref[...] = jnp.zeros_like(esum_ref)

    esum_ref[...] += jnp.broadcast_to(s, esum_ref.shape)


def _run_encoder(edge_attr, ett, w1, b1, w2, b2, lw):
    grid = (E // EB,)
    return pl.pallas_call(
        _enc_body,
        grid=grid,
        in_specs=[
            pl.BlockSpec((EB, 1 + ED), lambda i: (i, 0)),
            pl.BlockSpec((2, ED), lambda i: (0, 0)),
            pl.BlockSpec((ED, ED), lambda i: (0, 0)),
            pl.BlockSpec((1, ED), lambda i: (0, 0)),
            pl.BlockSpec((ED, ED), lambda i: (0, 0)),
            pl.BlockSpec((1, ED), lambda i: (0, 0)),
            pl.BlockSpec((ED, HC), lambda i: (0, 0)),
        ],
        out_specs=[
            pl.BlockSpec((EB, HC), lambda i: (i, 0)),
            pl.BlockSpec((8, ED), lambda i: (0, 0)),
        ],
        out_shape=[
            jax.ShapeDtypeStruct((E, HC), jnp.float32),
            jax.ShapeDtypeStruct((8, ED), jnp.float32),
        ],
    )(edge_attr, ett, w1, b1, w2, b2, lw)


def _proj_body(x_ref, wl_ref, wr_ref, xl_ref, xr_ref):
    x = x_ref[...]
    xl_ref[...] = jnp.dot(x, wl_ref[...], preferred_element_type=jnp.float32)
    xr_ref[...] = jnp.dot(x, wr_ref[...], preferred_element_type=jnp.float32)


def _run_proj(x, wl, wr):
    grid = (N // NB,)
    return pl.pallas_call(
        _proj_body,
        grid=grid,
        in_specs=[
            pl.BlockSpec((NB, D_IN), lambda i: (i, 0)),
            pl.BlockSpec((D_IN, HC), lambda i: (0, 0)),
            pl.BlockSpec((D_IN, HC), lambda i: (0, 0)),
        ],
        out_specs=[
            pl.BlockSpec((NB, HC), lambda i: (i, 0)),
            pl.BlockSpec((NB, HC), lambda i: (i, 0)),
        ],
        out_shape=[
            jax.ShapeDtypeStruct((N, HC), jnp.float32),
            jax.ShapeDtypeStruct((N, HC), jnp.float32),
        ],
    )(x, wl, wr)


def _take16(v, idx):
    return lax.gather(
        v, idx[:, None],
        dimension_numbers=lax.GatherDimensionNumbers(
            offset_dims=(), collapsed_slice_dims=(0,), start_index_map=(0,)),
        slice_sizes=(1,),
        mode=lax.GatherScatterMode.PROMISE_IN_BOUNDS)

def _sc_body(src_hbm, dst_hbm, xl_hbm, xr_hbm, ea_hbm, att_hbm, zacc_hbm,
             acc_out, den_out,
             src_v, dst_v, dstrow_v, xl_v, xr_v, ea_v, msg_v, den_v, att_v,
             acc_sh, den_sh, sem):
    cid = lax.axis_index("c")
    sid = lax.axis_index("s")
    wid = sid * NC + cid

    pltpu.sync_copy(att_hbm, att_v)
    pltpu.sync_copy(zacc_hbm.at[pl.ds(sid * RPS, RPS)], acc_sh.at[pl.ds(sid * RPS, RPS)])

    @pl.when(sid == NS - 1)
    def _():
        pltpu.sync_copy(zacc_hbm.at[pl.ds(TAIL0, TAIL)], acc_sh.at[pl.ds(TAIL0, TAIL)])

    @pl.when(sid != NS - 1)
    def _():
        pltpu.sync_copy(zacc_hbm.at[pl.ds(0, DRPS)], den_sh.at[pl.ds(sid * DRPS, DRPS)])

    @pl.when(sid == NS - 1)
    def _():
        pltpu.sync_copy(zacc_hbm.at[pl.ds(0, DTAIL)], den_sh.at[pl.ds(DTAIL0, DTAIL)])

    plsc.subcore_barrier()

    attc = [att_v[0, pl.ds(c * 16, 16)] for c in range(8)]
    lane = lax.iota(jnp.int32, 16)
    _zeros = jnp.full((16,), 0.0, jnp.float32)
    _i0 = jnp.full((16,), 0, jnp.int32)
    _i8 = jnp.full((16,), 8, jnp.int32)
    _alt = (lane & 1) * 8
    base0 = wid * EPW

    for e0 in range(K):
        for c0 in range(8):
            den_v[e0, pl.ds(c0 * 16, 16)] = _zeros

    def chunk_body(k, carry):
        base = base0 + k * K
        ci1 = pltpu.async_copy(src_hbm.at[pl.ds(base, K)], src_v, sem)
        ci2 = pltpu.async_copy(dst_hbm.at[pl.ds(base, K)], dst_v, sem)
        ce = pltpu.async_copy(ea_hbm.at[pl.ds(base, K)], ea_v, sem)
        ci1.wait()
        ci2.wait()
        c1 = pltpu.async_copy(xl_hbm.at[src_v], xl_v, sem)
        c2 = pltpu.async_copy(xr_hbm.at[dst_v], xr_v, sem)
        ce.wait()
        c1.wait()
        c2.wait()

        dvs = [dst_v[pl.ds(o, 16)] for o in (0, 16, 24)]
        for o, dvv in zip((0, 16, 24), dvs):
            dstrow_v[pl.ds(o, 16)] = lax.shift_right_logical(dvv, 3)
        offs = []
        for e in range(K):
            gi, go = (0, 0) if e < 16 else ((1, 16) if e < 32 else (2, 24))
            dv_e = dvs[gi]
            xlc = [xl_v[e, pl.ds(c * 16, 16)] for c in range(8)]
            hv = []
            for h in range(HEADS):
                ha = None
                for j in range(2):
                    c = 2 * h + j
                    t = xlc[c] + xr_v[e, pl.ds(c * 16, 16)] + ea_v[e, pl.ds(c * 16, 16)]
                    t = jnp.maximum(t, 0.2 * t)
                    t = t * attc[c]
                    ha = t if ha is None else ha + t
                hv.append(ha + _take16(ha, lane ^ 8))
            w01 = jnp.where(lane < 8, hv[0], hv[1])
            w23 = jnp.where(lane < 8, hv[2], hv[3])
            for m in (4, 2, 1):
                w01 = w01 + _take16(w01, lane ^ m)
                w23 = w23 + _take16(w23, lane ^ m)
            ep01 = jnp.exp(w01)
            ep23 = jnp.exp(w23)
            pv = [_take16(ep01, _i0), _take16(ep01, _i8),
                  _take16(ep23, _i0), _take16(ep23, _i8)]
            for c in range(8):
                msg_v[e, pl.ds(c * 16, 16)] = xlc[c] * pv[c // 2]
            d = dv_e[e - go]
            d01 = _take16(ep01, _alt)
            d23 = _take16(ep23, _alt)
            den16 = jnp.where(lane < 4, jnp.where(lane < 2, d01, d23), _zeros)
            off = (d & 7) * 16
            offs.append(off)
            den_v[e, pl.ds(off, 16)] = den16
        pltpu.sync_copy(msg_v, acc_sh.at[dst_v], add=True)
        pltpu.sync_copy(den_v, den_sh.at[dstrow_v], add=True)
        for e in range(K):
            den_v[e, pl.ds(offs[e], 16)] = _zeros
        return carry

    lax.fori_loop(0, NCHUNK, chunk_body, 0)
    plsc.subcore_barrier()

    pltpu.sync_copy(acc_sh.at[pl.ds(sid * RPS, RPS)],
                    acc_out.at[cid, pl.ds(sid * RPS, RPS)])

    @pl.when(sid == NS - 1)
    def _():
        pltpu.sync_copy(acc_sh.at[pl.ds(TAIL0, TAIL)], acc_out.at[cid, pl.ds(TAIL0, TAIL)])

    @pl.when(sid != NS - 1)
    def _():
        pltpu.sync_copy(den_sh.at[pl.ds(sid * DRPS, DRPS)],
                        den_out.at[cid, pl.ds(sid * DRPS, DRPS)])

    @pl.when(sid == NS - 1)
    def _():
        pltpu.sync_copy(den_sh.at[pl.ds(DTAIL0, DTAIL)], den_out.at[cid, pl.ds(DTAIL0, DTAIL)])


def _run_sc(src, dst, xl, xr, ea, att1):
    mesh = plsc.VectorSubcoreMesh(core_axis_name="c", subcore_axis_name="s")
    zacc = jnp.zeros((N, HC), jnp.float32)
    fn = functools.partial(
        pl.kernel,
        mesh=mesh,
        out_type=[
            jax.ShapeDtypeStruct((NC, N, HC), jnp.float32),
            jax.ShapeDtypeStruct((NC, DR, HC), jnp.float32),
        ],
        scratch_types=[
            pltpu.VMEM((K,), jnp.int32),
            pltpu.VMEM((K,), jnp.int32),
            pltpu.VMEM((K,), jnp.int32),
            pltpu.VMEM((K, HC), jnp.float32),
            pltpu.VMEM((K, HC), jnp.float32),
            pltpu.VMEM((K, HC), jnp.float32),
            pltpu.VMEM((K, HC), jnp.float32),
            pltpu.VMEM((K, HC), jnp.float32),
            pltpu.VMEM((1, HC), jnp.float32),
            pltpu.VMEM_SHARED((N, HC), jnp.float32),
            pltpu.VMEM_SHARED((DR, HC), jnp.float32),
            pltpu.SemaphoreType.DMA,
        ],
    )(_sc_body)
    return fn(src, dst, xl, xr, ea, att1, zacc)


def _merge_body(acc0_ref, acc1_ref, den0_ref, den1_ref, xl_ref, xr_ref,
                esum_ref, lw_ref, att_ref, bias_ref, out_ref):
    s = esum_ref[0:1, :] * jnp.float32(1.0 / E)
    r0 = jnp.dot(s, lw_ref[...], preferred_element_type=jnp.float32)
    xl = xl_ref[...]
    t = xl + xr_ref[...] + r0
    t = jnp.maximum(t, 0.2 * t)
    ta = t * att_ref[...]
    i0 = lax.broadcasted_iota(jnp.int32, (HC, HEADS), 0) // C_OUT
    h0 = lax.broadcasted_iota(jnp.int32, (HC, HEADS), 1)
    sel = (i0 == h0).astype(jnp.float32)
    i1 = lax.broadcasted_iota(jnp.int32, (HEADS, HC), 0)
    h1 = lax.broadcasted_iota(jnp.int32, (HEADS, HC), 1) // C_OUT
    selT = (i1 == h1).astype(jnp.float32)
    a4 = jnp.dot(ta, sel, preferred_element_type=jnp.float32)
    p4 = jnp.exp(a4)
    p128 = jnp.dot(p4, selT, preferred_element_type=jnp.float32)
    num = acc0_ref[...] + acc1_ref[...] + p128 * xl
    d4 = den0_ref[...] + den1_ref[...] + p4
    d128 = jnp.dot(d4, selT, preferred_element_type=jnp.float32)
    out_ref[...] = num / (d128 + 1e-16) + bias_ref[...]


def _run_merge(acc0, acc1, den0, den1, xl, xr, esum, lw, att2, bias2):
    grid = (N // NB,)
    return pl.pallas_call(
        _merge_body,
        grid=grid,
        in_specs=[
            pl.BlockSpec((NB, HC), lambda i: (i, 0)),
            pl.BlockSpec((NB, HC), lambda i: (i, 0)),
            pl.BlockSpec((NB, HEADS), lambda i: (i, 0)),
            pl.BlockSpec((NB, HEADS), lambda i: (i, 0)),
            pl.BlockSpec((NB, HC), lambda i: (i, 0)),
            pl.BlockSpec((NB, HC), lambda i: (i, 0)),
            pl.BlockSpec((8, ED), lambda i: (0, 0)),
            pl.BlockSpec((ED, HC), lambda i: (0, 0)),
            pl.BlockSpec((1, HC), lambda i: (0, 0)),
            pl.BlockSpec((1, HC), lambda i: (0, 0)),
        ],
        out_specs=pl.BlockSpec((NB, HC), lambda i: (i, 0)),
        out_shape=jax.ShapeDtypeStruct((N, HC), jnp.float32),
    )(acc0, acc1, den0, den1, xl, xr, esum, lw, att2, bias2)


def kernel(x, edge_index, edge_attr, edge_type_table, enc_W1, enc_b1, enc_W2, enc_b2,
           W_l, W_r, lin_edge_W, att, bias):
    ea, esum = _run_encoder(edge_attr, edge_type_table,
                            enc_W1, enc_b1.reshape(1, ED),
                            enc_W2, enc_b2.reshape(1, ED), lin_edge_W)
    xl, xr = _run_proj(x, W_l, W_r)
    src = edge_index[0]
    dst = edge_index[1]
    acc, den = _run_sc(src, dst, xl, xr, ea, att.reshape(1, HC))
    d4 = den[:, :N // 8].reshape(NC, N // 8, 8, 16)[..., :HEADS].reshape(NC, N, HEADS)
    out = _run_merge(acc[0], acc[1], d4[0], d4[1], xl, xr, esum,
                     lin_edge_W, att.reshape(1, HC), bias.reshape(1, HC))
    return out


# speedup vs baseline: 1.2441x; 1.0038x over previous
import functools

import jax
import jax.numpy as jnp
from jax import lax
from jax.experimental import pallas as pl
from jax.experimental.pallas import tpu as pltpu
from jax.experimental.pallas import tpu_sc as plsc

N = 10000
E = 320000
D_IN = 128
HEADS = 4
C_OUT = 32
HC = HEADS * C_OUT
ED = 16

NC = 2
NS = 16
NW = NC * NS
EPW = E // NW
K = 40
NCHUNK = EPW // K
RPS = 624
TAIL = N - RPS * NS
TAIL0 = RPS * NS

DR = 1256
DRPS = 80
DTAIL = DR - DRPS * (NS - 1) + 0
DTAIL0 = DRPS * (NS - 1)
DTAIL = DR - DTAIL0

EB = 8000
NB = 2000


def _enc_body(attr_ref, ett_ref, w1_ref, b1_ref, w2_ref, b2_ref, lw_ref,
              ea_ref, esum_ref):
    blk = attr_ref[...]
    t0 = blk[:, 0:1]
    cond = t0 < 1.0
    ete = jnp.where(cond, ett_ref[0:1, :], ett_ref[1:2, :])
    feat = blk[:, 1:1 + ED] + ete
    h = jnp.maximum(jnp.dot(feat, w1_ref[...], preferred_element_type=jnp.float32)
                    + b1_ref[...], 0.0)
    enh = jnp.dot(h, w2_ref[...], preferred_element_type=jnp.float32) + b2_ref[...]
    ea_ref[...] = jnp.dot(enh, lw_ref[...], preferred_element_type=jnp.float32)
    s = jnp.sum(enh, axis=0, keepdims=True)

    @pl.when(pl.program_id(0) == 0)
    def _():
        esum_ref[...] = jnp.zeros_like(esum_ref)

    esum_ref[...] += jnp.broadcast_to(s, esum_ref.shape)


def _run_encoder(edge_attr, ett, w1, b1, w2, b2, lw):
    grid = (E // EB,)
    return pl.pallas_call(
        _enc_body,
        grid=grid,
        in_specs=[
            pl.BlockSpec((EB, 1 + ED), lambda i: (i, 0)),
            pl.BlockSpec((2, ED), lambda i: (0, 0)),
            pl.BlockSpec((ED, ED), lambda i: (0, 0)),
            pl.BlockSpec((1, ED), lambda i: (0, 0)),
            pl.BlockSpec((ED, ED), lambda i: (0, 0)),
            pl.BlockSpec((1, ED), lambda i: (0, 0)),
            pl.BlockSpec((ED, HC), lambda i: (0, 0)),
        ],
        out_specs=[
            pl.BlockSpec((EB, HC), lambda i: (i, 0)),
            pl.BlockSpec((8, ED), lambda i: (0, 0)),
        ],
        out_shape=[
            jax.ShapeDtypeStruct((E, HC), jnp.float32),
            jax.ShapeDtypeStruct((8, ED), jnp.float32),
        ],
    )(edge_attr, ett, w1, b1, w2, b2, lw)


def _proj_body(x_ref, wl_ref, wr_ref, xl_ref, xr_ref):
    x = x_ref[...]
    xl_ref[...] = jnp.dot(x, wl_ref[...], preferred_element_type=jnp.float32)
    xr_ref[...] = jnp.dot(x, wr_ref[...], preferred_element_type=jnp.float32)


def _run_proj(x, wl, wr):
    grid = (N // NB,)
    return pl.pallas_call(
        _proj_body,
        grid=grid,
        in_specs=[
            pl.BlockSpec((NB, D_IN), lambda i: (i, 0)),
            pl.BlockSpec((D_IN, HC), lambda i: (0, 0)),
            pl.BlockSpec((D_IN, HC), lambda i: (0, 0)),
        ],
        out_specs=[
            pl.BlockSpec((NB, HC), lambda i: (i, 0)),
            pl.BlockSpec((NB, HC), lambda i: (i, 0)),
        ],
        out_shape=[
            jax.ShapeDtypeStruct((N, HC), jnp.float32),
            jax.ShapeDtypeStruct((N, HC), jnp.float32),
        ],
    )(x, wl, wr)


def _take16(v, idx):
    return lax.gather(
        v, idx[:, None],
        dimension_numbers=lax.GatherDimensionNumbers(
            offset_dims=(), collapsed_slice_dims=(0,), start_index_map=(0,)),
        slice_sizes=(1,),
        mode=lax.GatherScatterMode.PROMISE_IN_BOUNDS)

def _sc_body(src_hbm, dst_hbm, xl_hbm, xr_hbm, ea_hbm, att_hbm, zacc_hbm,
             acc_out, den_out,
             src_v, dst_v, dstrow_v, xl_v, xr_v, ea_v, msg_v, den_v, att_v,
             acc_sh, den_sh, sem):
    cid = lax.axis_index("c")
    sid = lax.axis_index("s")
    wid = sid * NC + cid

    pltpu.sync_copy(att_hbm, att_v)
    pltpu.sync_copy(zacc_hbm.at[pl.ds(sid * RPS, RPS)], acc_sh.at[pl.ds(sid * RPS, RPS)])

    @pl.when(sid == NS - 1)
    def _():
        pltpu.sync_copy(zacc_hbm.at[pl.ds(TAIL0, TAIL)], acc_sh.at[pl.ds(TAIL0, TAIL)])

    @pl.when(sid != NS - 1)
    def _():
        pltpu.sync_copy(zacc_hbm.at[pl.ds(0, DRPS)], den_sh.at[pl.ds(sid * DRPS, DRPS)])

    @pl.when(sid == NS - 1)
    def _():
        pltpu.sync_copy(zacc_hbm.at[pl.ds(0, DTAIL)], den_sh.at[pl.ds(DTAIL0, DTAIL)])

    plsc.subcore_barrier()

    attc = [att_v[0, pl.ds(c * 16, 16)] for c in range(8)]
    lane = lax.iota(jnp.int32, 16)
    _zeros = jnp.full((16,), 0.0, jnp.float32)
    _i0 = jnp.full((16,), 0, jnp.int32)
    _i8 = jnp.full((16,), 8, jnp.int32)
    _alt = (lane & 1) * 8
    base0 = wid * EPW

    for e0 in range(K):
        for c0 in range(8):
            den_v[e0, pl.ds(c0 * 16, 16)] = _zeros

    def chunk_body(k, carry):
        base = base0 + k * K
        ci1 = pltpu.async_copy(src_hbm.at[pl.ds(base, K)], src_v, sem)
        ci2 = pltpu.async_copy(dst_hbm.at[pl.ds(base, K)], dst_v, sem)
        ce = pltpu.async_copy(ea_hbm.at[pl.ds(base, K)], ea_v, sem)
        ci1.wait()
        ci2.wait()
        c1 = pltpu.async_copy(xl_hbm.at[src_v], xl_v, sem)
        c2 = pltpu.async_copy(xr_hbm.at[dst_v], xr_v, sem)
        ce.wait()
        c1.wait()
        c2.wait()

        dvs = [dst_v[pl.ds(o, 16)] for o in (0, 16, 24)]
        for o, dvv in zip((0, 16, 24), dvs):
            dstrow_v[pl.ds(o, 16)] = lax.shift_right_logical(dvv, 3)
        offs = []
        for e in range(K):
            gi, go = (0, 0) if e < 16 else ((1, 16) if e < 32 else (2, 24))
            dv_e = dvs[gi]
            xlc = [xl_v[e, pl.ds(c * 16, 16)] for c in range(8)]
            hv = []
            for h in range(HEADS):
                ha = None
                for j in range(2):
                    c = 2 * h + j
                    t = xlc[c] + xr_v[e, pl.ds(c * 16, 16)] + ea_v[e, pl.ds(c * 16, 16)]
                    t = jnp.maximum(t, 0.2 * t)
                    t = t * attc[c]
                    ha = t if ha is None else ha + t
                hv.append(ha + _take16(ha, lane ^ 8))
            w01 = jnp.where(lane < 8, hv[0], hv[1])
            w23 = jnp.where(lane < 8, hv[2], hv[3])
            for m in (4, 2, 1):
                w01 = w01 + _take16(w01, lane ^ m)
                w23 = w23 + _take16(w23, lane ^ m)
            ep01 = jnp.exp(w01)
            ep23 = jnp.exp(w23)
            pv = [_take16(ep01, _i0), _take16(ep01, _i8),
                  _take16(ep23, _i0), _take16(ep23, _i8)]
            for c in range(8):
                msg_v[e, pl.ds(c * 16, 16)] = xlc[c] * pv[c // 2]
            d = dv_e[e - go]
            d01 = _take16(ep01, _alt)
            d23 = _take16(ep23, _alt)
            den16 = jnp.where(lane < 4, jnp.where(lane < 2, d01, d23), _zeros)
            off = (d & 7) * 16
            offs.append(off)
            den_v[e, pl.ds(off, 16)] = den16
        s1 = pltpu.async_copy(msg_v, acc_sh.at[dst_v], sem, add=True)
        s2 = pltpu.async_copy(den_v, den_sh.at[dstrow_v], sem, add=True)
        s1.wait()
        s2.wait()
        for e in range(K):
            den_v[e, pl.ds(offs[e], 16)] = _zeros
        return carry

    lax.fori_loop(0, NCHUNK, chunk_body, 0)
    plsc.subcore_barrier()

    pltpu.sync_copy(acc_sh.at[pl.ds(sid * RPS, RPS)],
                    acc_out.at[cid, pl.ds(sid * RPS, RPS)])

    @pl.when(sid == NS - 1)
    def _():
        pltpu.sync_copy(acc_sh.at[pl.ds(TAIL0, TAIL)], acc_out.at[cid, pl.ds(TAIL0, TAIL)])

    @pl.when(sid != NS - 1)
    def _():
        pltpu.sync_copy(den_sh.at[pl.ds(sid * DRPS, DRPS)],
                        den_out.at[cid, pl.ds(sid * DRPS, DRPS)])

    @pl.when(sid == NS - 1)
    def _():
        pltpu.sync_copy(den_sh.at[pl.ds(DTAIL0, DTAIL)], den_out.at[cid, pl.ds(DTAIL0, DTAIL)])


def _run_sc(src, dst, xl, xr, ea, att1):
    mesh = plsc.VectorSubcoreMesh(core_axis_name="c", subcore_axis_name="s")
    zacc = jnp.zeros((N, HC), jnp.float32)
    fn = functools.partial(
        pl.kernel,
        mesh=mesh,
        out_type=[
            jax.ShapeDtypeStruct((NC, N, HC), jnp.float32),
            jax.ShapeDtypeStruct((NC, DR, HC), jnp.float32),
        ],
        scratch_types=[
            pltpu.VMEM((K,), jnp.int32),
            pltpu.VMEM((K,), jnp.int32),
            pltpu.VMEM((K,), jnp.int32),
            pltpu.VMEM((K, HC), jnp.float32),
            pltpu.VMEM((K, HC), jnp.float32),
            pltpu.VMEM((K, HC), jnp.float32),
            pltpu.VMEM((K, HC), jnp.float32),
            pltpu.VMEM((K, HC), jnp.float32),
            pltpu.VMEM((1, HC), jnp.float32),
            pltpu.VMEM_SHARED((N, HC), jnp.float32),
            pltpu.VMEM_SHARED((DR, HC), jnp.float32),
            pltpu.SemaphoreType.DMA,
        ],
    )(_sc_body)
    return fn(src, dst, xl, xr, ea, att1, zacc)


def _merge_body(acc0_ref, acc1_ref, den0_ref, den1_ref, xl_ref, xr_ref,
                esum_ref, lw_ref, att_ref, bias_ref, out_ref):
    s = esum_ref[0:1, :] * jnp.float32(1.0 / E)
    r0 = jnp.dot(s, lw_ref[...], preferred_element_type=jnp.float32)
    xl = xl_ref[...]
    t = xl + xr_ref[...] + r0
    t = jnp.maximum(t, 0.2 * t)
    ta = t * att_ref[...]
    i0 = lax.broadcasted_iota(jnp.int32, (HC, HEADS), 0) // C_OUT
    h0 = lax.broadcasted_iota(jnp.int32, (HC, HEADS), 1)
    sel = (i0 == h0).astype(jnp.float32)
    i1 = lax.broadcasted_iota(jnp.int32, (HEADS, HC), 0)
    h1 = lax.broadcasted_iota(jnp.int32, (HEADS, HC), 1) // C_OUT
    selT = (i1 == h1).astype(jnp.float32)
    a4 = jnp.dot(ta, sel, preferred_element_type=jnp.float32)
    p4 = jnp.exp(a4)
    p128 = jnp.dot(p4, selT, preferred_element_type=jnp.float32)
    num = acc0_ref[...] + acc1_ref[...] + p128 * xl
    d4 = den0_ref[...] + den1_ref[...] + p4
    d128 = jnp.dot(d4, selT, preferred_element_type=jnp.float32)
    out_ref[...] = num / (d128 + 1e-16) + bias_ref[...]


def _run_merge(acc0, acc1, den0, den1, xl, xr, esum, lw, att2, bias2):
    grid = (N // NB,)
    return pl.pallas_call(
        _merge_body,
        grid=grid,
        in_specs=[
            pl.BlockSpec((NB, HC), lambda i: (i, 0)),
            pl.BlockSpec((NB, HC), lambda i: (i, 0)),
            pl.BlockSpec((NB, HEADS), lambda i: (i, 0)),
            pl.BlockSpec((NB, HEADS), lambda i: (i, 0)),
            pl.BlockSpec((NB, HC), lambda i: (i, 0)),
            pl.BlockSpec((NB, HC), lambda i: (i, 0)),
            pl.BlockSpec((8, ED), lambda i: (0, 0)),
            pl.BlockSpec((ED, HC), lambda i: (0, 0)),
            pl.BlockSpec((1, HC), lambda i: (0, 0)),
            pl.BlockSpec((1, HC), lambda i: (0, 0)),
        ],
        out_specs=pl.BlockSpec((NB, HC), lambda i: (i, 0)),
        out_shape=jax.ShapeDtypeStruct((N, HC), jnp.float32),
    )(acc0, acc1, den0, den1, xl, xr, esum, lw, att2, bias2)


def kernel(x, edge_index, edge_attr, edge_type_table, enc_W1, enc_b1, enc_W2, enc_b2,
           W_l, W_r, lin_edge_W, att, bias):
    ea, esum = _run_encoder(edge_attr, edge_type_table,
                            enc_W1, enc_b1.reshape(1, ED),
                            enc_W2, enc_b2.reshape(1, ED), lin_edge_W)
    xl, xr = _run_proj(x, W_l, W_r)
    src = edge_index[0]
    dst = edge_index[1]
    acc, den = _run_sc(src, dst, xl, xr, ea, att.reshape(1, HC))
    d4 = den[:, :N // 8].reshape(NC, N // 8, 8, 16)[..., :HEADS].reshape(NC, N, HEADS)
    out = _run_merge(acc[0], acc[1], d4[0], d4[1], xl, xr, esum,
                     lin_edge_W, att.reshape(1, HC), bias.reshape(1, HC))
    return out
